# baseline (speedup 1.0000x reference)
import dataclasses

import jax
import jax.numpy as jnp
from jax import lax
from jax.experimental import pallas as pl
from jax.experimental.pallas import tpu as pltpu
from jax.experimental.pallas import tpu_sc as plsc

N = 10000
NP = 10240
D = 128
NC = 2
NS = 16
L = 16
NW = NC * NS
EPT = 10240
E_PAD = NW * EPT
BLK = 128
RPT = NP // NS
GB = 1024
GRID = NP // GB

_mesh = plsc.VectorSubcoreMesh(core_axis_name="c", subcore_axis_name="s")

_sc_params = pltpu.CompilerParams()
if "needs_layout_passes" in pltpu.CompilerParams.__dataclass_fields__:
    _sc_params = dataclasses.replace(_sc_params, needs_layout_passes=False)


def _pre_body(x_ref, w_ref, o_ref):
    o_ref[...] = jnp.dot(x_ref[...], w_ref[...],
                         preferred_element_type=jnp.float32)


def _tc_pre(x, wcat):
    return pl.pallas_call(
        _pre_body,
        grid=(GRID,),
        in_specs=[pl.BlockSpec((GB, D), lambda i: (i, 0)),
                  pl.BlockSpec((D, 3 * D), lambda i: (0, 0))],
        out_specs=pl.BlockSpec((GB, 3 * D), lambda i: (i, 0)),
        out_shape=jax.ShapeDtypeStruct((NP, 3 * D), jnp.float32),
    )(x, wcat)


def _mid_body(cntp_ref, xw_ref, u_ref):
    cnt = jnp.sum(cntp_ref[...], axis=0)
    dinv = lax.rsqrt(cnt + 1.0)
    u_ref[...] = dinv[:, None] * xw_ref[...]


def _tc_mid(cnt_parts, xw):
    return pl.pallas_call(
        _mid_body,
        grid=(GRID,),
        in_specs=[pl.BlockSpec((NW, GB), lambda i: (0, i)),
                  pl.BlockSpec((GB, D), lambda i: (i, 0))],
        out_specs=pl.BlockSpec((GB, D), lambda i: (i, 0)),
        out_shape=jax.ShapeDtypeStruct((NP, D), jnp.float32),
    )(cnt_parts, xw)


def _post_body(cntp_ref, asump_ref, ssage_ref, sgcn_ref, sgat_ref, x_ref,
               xwg_ref, wsl_ref, wsr_ref, wfus_ref, bg_ref, bga_ref, bsl_ref,
               bf_ref, o_ref):
    cnt = jnp.sum(cntp_ref[...], axis=0)
    asum_e = jnp.sum(asump_ref[...], axis=0)
    s_sage = ssage_ref[0] + ssage_ref[1]
    s_gcn = sgcn_ref[0] + sgcn_ref[1]
    s_gat = sgat_ref[0] + sgat_ref[1]
    xwg = xwg_ref[...]
    xw = xwg[:, 0:D]
    xg = xwg[:, D:2 * D]
    a_s = xwg[:, 2 * D:2 * D + 1]
    a_d = xwg[:, 2 * D + 1:2 * D + 2]

    dinv = lax.rsqrt(cnt + 1.0)[:, None]
    h_gcn = jnp.maximum(dinv * s_gcn + dinv * dinv * xw + bg_ref[...], 0.0)

    al = a_s + a_d
    ae_self = jnp.exp(jnp.maximum(al, 0.2 * al))
    denom = asum_e[:, None] + ae_self + 1e-16
    h_gat = jnp.maximum((s_gat + ae_self * xg) / denom + bga_ref[...], 0.0)

    mean = s_sage / jnp.maximum(cnt, 1.0)[:, None]
    h_sage = jnp.maximum(
        jnp.dot(mean, wsl_ref[...], preferred_element_type=jnp.float32)
        + bsl_ref[...]
        + jnp.dot(x_ref[...], wsr_ref[...], preferred_element_type=jnp.float32),
        0.0)

    wfus = wfus_ref[...]
    o_ref[...] = (
        jnp.dot(h_gcn, wfus[0:D], preferred_element_type=jnp.float32)
        + jnp.dot(h_gat, wfus[D:2 * D], preferred_element_type=jnp.float32)
        + jnp.dot(h_sage, wfus[2 * D:3 * D], preferred_element_type=jnp.float32)
        + bf_ref[...])


def _tc_post(cnt_parts, asum_parts, s_sage, s_gcn, s_gat, x, xwg,
             W_sage_l, W_sage_r, W_fus, b_gcn, b_gat, b_sage_l, b_fus):
    return pl.pallas_call(
        _post_body,
        grid=(GRID,),
        in_specs=[
            pl.BlockSpec((NW, GB), lambda i: (0, i)),
            pl.BlockSpec((NW, GB), lambda i: (0, i)),
            pl.BlockSpec((NC, GB, D), lambda i: (0, i, 0)),
            pl.BlockSpec((NC, GB, D), lambda i: (0, i, 0)),
            pl.BlockSpec((NC, GB, D), lambda i: (0, i, 0)),
            pl.BlockSpec((GB, D), lambda i: (i, 0)),
            pl.BlockSpec((GB, 3 * D), lambda i: (i, 0)),
            pl.BlockSpec((D, D), lambda i: (0, 0)),
            pl.BlockSpec((D, D), lambda i: (0, 0)),
            pl.BlockSpec((3 * D, D), lambda i: (0, 0)),
            pl.BlockSpec((1, D), lambda i: (0, 0)),
            pl.BlockSpec((1, D), lambda i: (0, 0)),
            pl.BlockSpec((1, D), lambda i: (0, 0)),
            pl.BlockSpec((1, D), lambda i: (0, 0)),
        ],
        out_specs=pl.BlockSpec((GB, D), lambda i: (i, 0)),
        out_shape=jax.ShapeDtypeStruct((NP, D), jnp.float32),
    )(cnt_parts, asum_parts, s_sage, s_gcn, s_gat, x, xwg,
      W_sage_l, W_sage_r, W_fus,
      b_gcn.reshape(1, D), b_gat.reshape(1, D), b_sage_l.reshape(1, D),
      b_fus.reshape(1, D))


def _sc0_body(row_hbm, col_hbm, asrc_hbm, adst_hbm,
              ae_hbm, asum_hbm, cnt_hbm,
              asrc_v, adst_v, row_v, col_v, ae_v, asum_v, cnt_v):
    c = lax.axis_index("c")
    s = lax.axis_index("s")
    wid = s * NC + c
    base = wid * EPT

    pltpu.sync_copy(asrc_hbm, asrc_v)
    pltpu.sync_copy(adst_hbm, adst_v)
    pltpu.sync_copy(row_hbm.at[pl.ds(base, EPT)], row_v)
    pltpu.sync_copy(col_hbm.at[pl.ds(base, EPT)], col_v)

    zero16 = jnp.zeros((L,), jnp.float32)

    @pl.loop(0, NP, step=L)
    def _(i):
        asum_v[pl.ds(i, L)] = zero16
        cnt_v[pl.ds(i, L)] = zero16

    ones = jnp.ones((L,), jnp.float32)

    @pl.loop(0, EPT, step=L)
    def _(i):
        r = row_v[pl.ds(i, L)]
        cc = col_v[pl.ds(i, L)]
        sa = plsc.load_gather(asrc_v, [r])
        da = plsc.load_gather(adst_v, [cc])
        al = sa + da
        ae = jnp.exp(jnp.maximum(al, 0.2 * al))
        ae_v[pl.ds(i, L)] = ae
        plsc.addupdate_scatter(asum_v, [cc], ae)
        plsc.addupdate_scatter(cnt_v, [cc], ones)

    pltpu.sync_copy(ae_v, ae_hbm.at[pl.ds(base, EPT)])
    pltpu.sync_copy(asum_v, asum_hbm.at[wid])
    pltpu.sync_copy(cnt_v, cnt_hbm.at[wid])


def _sc_edge_scalars(row, col, a_src, a_dst):
    kern = pl.kernel(
        _sc0_body,
        out_type=(jax.ShapeDtypeStruct((E_PAD,), jnp.float32),
                  jax.ShapeDtypeStruct((NW, NP), jnp.float32),
                  jax.ShapeDtypeStruct((NW, NP), jnp.float32)),
        mesh=_mesh,
        scratch_types=[
            pltpu.VMEM((NP,), jnp.float32),
            pltpu.VMEM((NP,), jnp.float32),
            pltpu.VMEM((EPT,), jnp.int32),
            pltpu.VMEM((EPT,), jnp.int32),
            pltpu.VMEM((EPT,), jnp.float32),
            pltpu.VMEM((NP,), jnp.float32),
            pltpu.VMEM((NP,), jnp.float32),
        ],
        compiler_params=_sc_params,
    )
    return kern(row, col, a_src, a_dst)


FBLK = 64
FNSTEP = EPT // FBLK
NBUF = 4
NRING = 8
DRB = 64


def _agg_bodies(*refs):
    (tabx_hbm, tabu_hbm, tabg_hbm, row_hbm, col_hbm, ae_hbm,
     outx_hbm, outu_hbm, outg_hbm, acc_sh,
     rowr, colr, aer, b0, b1, b2, b3, zb,
     is0, is1, is2, is3, is4, is5, is6, is7,
     gs0, gs1, gs2, gs3, ss0, ss1, ss2, ss3) = refs
    c = lax.axis_index("c")
    s = lax.axis_index("s")
    wid = s * NC + c
    sbase = wid * FNSTEP
    rbase = s * RPT
    isems = (is0, is1, is2, is3, is4, is5, is6, is7)
    bufs = (b0, b1, b2, b3)
    gsems = (gs0, gs1, gs2, gs3)
    ssems = (ss0, ss1, ss2, ss3)

    zero16 = jnp.zeros((L,), jnp.float32)

    def start_idx(step, slot, scaled):
        pltpu.async_copy(row_hbm.at[sbase + step], rowr.at[slot], isems[slot])
        pltpu.async_copy(col_hbm.at[sbase + step], colr.at[slot], isems[slot])
        if scaled:
            pltpu.async_copy(ae_hbm.at[sbase + step], aer.at[slot],
                             isems[slot])

    def wait_idx(slot, scaled):
        pltpu.make_async_copy(row_hbm.at[0], rowr.at[slot], isems[slot]).wait()
        pltpu.make_async_copy(col_hbm.at[0], colr.at[slot], isems[slot]).wait()
        if scaled:
            pltpu.make_async_copy(ae_hbm.at[0], aer.at[slot],
                                  isems[slot]).wait()

    def start_gather(tab_hbm, slot, pb):
        pltpu.async_copy(tab_hbm.at[rowr.at[slot]], bufs[pb], gsems[pb])

    def wait_gather(pb):
        pltpu.make_async_copy(tabx_hbm.at[pl.ds(0, FBLK)], bufs[pb],
                              gsems[pb]).wait()

    def wait_scatter(pb):
        pltpu.make_async_copy(tabx_hbm.at[pl.ds(0, FBLK)], bufs[pb],
                              ssems[pb]).wait()

    def scale_rows(pb, slot):
        @pl.loop(0, FBLK, unroll=2)
        def _(j):
            a = plsc.load_gather(aer.at[slot],
                                 [jnp.full((L,), j, jnp.int32)])
            for k in range(D // L):
                sl = pl.ds(k * L, L)
                bufs[pb][j, sl] = bufs[pb][j, sl] * a

    def zero_acc():
        @pl.loop(0, DRB)
        def _(j):
            for k in range(D // L):
                zb[j, pl.ds(k * L, L)] = zero16

        @pl.loop(0, RPT, step=DRB)
        def _(r0):
            pltpu.sync_copy(zb, acc_sh.at[pl.ds(rbase + r0, DRB)])

    def prime(tab_hbm, scaled):
        for k in range(NBUF):
            start_idx(k, k, scaled)
        for k in range(NBUF - 1):
            wait_idx(k, scaled)
            start_gather(tab_hbm, k, k)

    def sections(tab_hbm, scaled):
        @pl.loop(0, FNSTEP, step=NRING)
        def _(g):
            for b in range(NRING):
                gi = g + b
                s8 = b
                b4 = b % NBUF
                wait_gather(b4)
                if scaled:
                    scale_rows(b4, s8)
                pltpu.async_copy(bufs[b4], acc_sh.at[colr.at[s8]],
                                 ssems[b4], add=True)

                @pl.when(gi + NBUF < FNSTEP)
                def _():
                    start_idx(gi + NBUF, (b + NBUF) % NRING, scaled)

                @pl.when(gi + 3 < FNSTEP)
                def _():
                    @pl.when(gi >= 1)
                    def _():
                        wait_scatter((b + 3) % NBUF)

                    wait_idx((b + 3) % NRING, scaled)
                    start_gather(tab_hbm, (b + 3) % NRING, (b + 3) % NBUF)

        for k in range(NBUF):
            wait_scatter(k)

    def drain_to(out_hbm):
        @pl.loop(0, RPT, step=DRB)
        def _(r0):
            pltpu.sync_copy(acc_sh.at[pl.ds(rbase + r0, DRB)], zb)
            pltpu.sync_copy(zb, out_hbm.at[c].at[pl.ds(rbase + r0, DRB)])

    phases = ((tabx_hbm, outx_hbm, False),
              (tabu_hbm, outu_hbm, False),
              (tabg_hbm, outg_hbm, True))
    for tab, out, scaled in phases:
        zero_acc()
        prime(tab, scaled)
        plsc.subcore_barrier()
        sections(tab, scaled)
        plsc.subcore_barrier()
        drain_to(out)


def _make_agg3():
    ot = jax.ShapeDtypeStruct((NC, NP, D), jnp.float32)
    scratch = [
        pltpu.VMEM_SHARED((NP, D), jnp.float32),
        pltpu.VMEM((NRING, FBLK), jnp.int32),
        pltpu.VMEM((NRING, FBLK), jnp.int32),
        pltpu.VMEM((NRING, FBLK), jnp.float32),
    ]
    scratch += [pltpu.VMEM((FBLK, D), jnp.float32)] * NBUF
    scratch += [pltpu.VMEM((DRB, D), jnp.float32)]
    scratch += [pltpu.SemaphoreType.DMA] * (NRING + 2 * NBUF)
    return pl.kernel(
        _agg_bodies,
        out_type=(ot, ot, ot),
        mesh=_mesh,
        scratch_types=scratch,
        compiler_params=_sc_params,
    )


_sc_agg3 = _make_agg3()


def kernel(x, edge_index, W_gcn, b_gcn, W_gat, att_src, att_dst, b_gat,
           W_sage_l, b_sage_l, W_sage_r, W_fus, b_fus):
    row = edge_index[0]
    col = edge_index[1]
    npad = E_PAD - row.shape[0]
    ar = jnp.arange(npad, dtype=jnp.int32)
    row_p = jnp.concatenate([row, (ar * 37) % N])
    col_p = jnp.concatenate([col, N + ar % (NP - N)])

    x_p = jnp.zeros((NP, D), jnp.float32).at[:N].set(x)
    att2 = (jnp.zeros((D, D), jnp.float32)
            .at[:, 0].set(W_gat @ att_src).at[:, 1].set(W_gat @ att_dst))
    wcat = jnp.concatenate([W_gcn, W_gat, att2], axis=1)

    xwg = _tc_pre(x_p, wcat)
    xw = xwg[:, 0:D]
    xg = xwg[:, D:2 * D]
    a_src = xwg[:, 2 * D]
    a_dst = xwg[:, 2 * D + 1]

    row2 = row_p.reshape(NW * FNSTEP, FBLK)
    col2 = col_p.reshape(NW * FNSTEP, FBLK)
    ae, asum_parts, cnt_parts = _sc_edge_scalars(row_p, col_p, a_src, a_dst)
    ae2 = ae.reshape(NW * FNSTEP, FBLK)
    u = _tc_mid(cnt_parts, xw)
    s_sage, s_gcn, s_gat = _sc_agg3(x_p, u, xg, row2, col2, ae2)

    out = _tc_post(cnt_parts, asum_parts, s_sage, s_gcn, s_gat, x_p, xwg,
                   W_sage_l, W_sage_r, W_fus, b_gcn, b_gat, b_sage_l, b_fus)
    return out[:N]


# speedup vs baseline: 1.0075x; 1.0075x over previous
import dataclasses

import jax
import jax.numpy as jnp
from jax import lax
from jax.experimental import pallas as pl
from jax.experimental.pallas import tpu as pltpu
from jax.experimental.pallas import tpu_sc as plsc

N = 10000
NP = 10240
D = 128
NC = 2
NS = 16
L = 16
NW = NC * NS
EPT = 10240
E_PAD = NW * EPT
BLK = 128
RPT = NP // NS
GB = 1024
GRID = NP // GB

_mesh = plsc.VectorSubcoreMesh(core_axis_name="c", subcore_axis_name="s")

_sc_params = pltpu.CompilerParams()
if "needs_layout_passes" in pltpu.CompilerParams.__dataclass_fields__:
    _sc_params = dataclasses.replace(_sc_params, needs_layout_passes=False)


def _pre_body(x_ref, wg_ref, wt_ref, avs_ref, avd_ref, wsr_ref,
              xw_o, xg_o, as_o, ad_o, xsr_o):
    xb = x_ref[...]
    xw_o[...] = jnp.dot(xb, wg_ref[...], preferred_element_type=jnp.float32)
    xg_o[...] = jnp.dot(xb, wt_ref[...], preferred_element_type=jnp.float32)
    as_o[...] = jnp.dot(xb, avs_ref[...], preferred_element_type=jnp.float32,
                        precision=lax.Precision.HIGHEST)
    ad_o[...] = jnp.dot(xb, avd_ref[...], preferred_element_type=jnp.float32,
                        precision=lax.Precision.HIGHEST)
    xsr_o[...] = jnp.dot(xb, wsr_ref[...], preferred_element_type=jnp.float32)


def _tc_pre(x, W_gcn, W_gat, av_s, av_d, W_sage_r):
    mat = jax.ShapeDtypeStruct((NP, D), jnp.float32)
    vec = jax.ShapeDtypeStruct((NP, 1), jnp.float32)
    return pl.pallas_call(
        _pre_body,
        grid=(GRID,),
        in_specs=[pl.BlockSpec((GB, D), lambda i: (i, 0)),
                  pl.BlockSpec((D, D), lambda i: (0, 0)),
                  pl.BlockSpec((D, D), lambda i: (0, 0)),
                  pl.BlockSpec((D, 1), lambda i: (0, 0)),
                  pl.BlockSpec((D, 1), lambda i: (0, 0)),
                  pl.BlockSpec((D, D), lambda i: (0, 0))],
        out_specs=[pl.BlockSpec((GB, D), lambda i: (i, 0)),
                   pl.BlockSpec((GB, D), lambda i: (i, 0)),
                   pl.BlockSpec((GB, 1), lambda i: (i, 0)),
                   pl.BlockSpec((GB, 1), lambda i: (i, 0)),
                   pl.BlockSpec((GB, D), lambda i: (i, 0))],
        out_shape=(mat, mat, vec, vec, mat),
    )(x, W_gcn, W_gat, av_s, av_d, W_sage_r)


def _mid_body(cntp_ref, xw_ref, u_ref):
    cnt = jnp.sum(cntp_ref[...], axis=0)
    dinv = lax.rsqrt(cnt + 1.0)
    u_ref[...] = dinv[:, None] * xw_ref[...]


def _tc_mid(cnt_parts, xw):
    return pl.pallas_call(
        _mid_body,
        grid=(GRID,),
        in_specs=[pl.BlockSpec((NW, GB), lambda i: (0, i)),
                  pl.BlockSpec((GB, D), lambda i: (i, 0))],
        out_specs=pl.BlockSpec((GB, D), lambda i: (i, 0)),
        out_shape=jax.ShapeDtypeStruct((NP, D), jnp.float32),
    )(cnt_parts, xw)


def _post_body(cntp_ref, asump_ref, ssage_ref, sgcn_ref, sgat_ref, xsr_ref,
               xw_ref, xg_ref, as_ref, ad_ref, wsl_ref, wfus_ref, bg_ref,
               bga_ref, bsl_ref, bf_ref, o_ref):
    cnt = jnp.sum(cntp_ref[...], axis=0)
    asum_e = jnp.sum(asump_ref[...], axis=0)
    s_sage = ssage_ref[0] + ssage_ref[1]
    s_gcn = sgcn_ref[0] + sgcn_ref[1]
    s_gat = sgat_ref[0] + sgat_ref[1]
    xw = xw_ref[...]
    xg = xg_ref[...]

    dinv = lax.rsqrt(cnt + 1.0)[:, None]
    h_gcn = jnp.maximum(dinv * s_gcn + dinv * dinv * xw + bg_ref[...], 0.0)

    al = as_ref[...] + ad_ref[...]
    ae_self = jnp.exp(jnp.maximum(al, 0.2 * al))
    denom = asum_e[:, None] + ae_self + 1e-16
    h_gat = jnp.maximum((s_gat + ae_self * xg) / denom + bga_ref[...], 0.0)

    mean = s_sage / jnp.maximum(cnt, 1.0)[:, None]
    h_sage = jnp.maximum(
        jnp.dot(mean, wsl_ref[...], preferred_element_type=jnp.float32)
        + bsl_ref[...] + xsr_ref[...],
        0.0)

    wfus = wfus_ref[...]
    o_ref[...] = (
        jnp.dot(h_gcn, wfus[0:D], preferred_element_type=jnp.float32)
        + jnp.dot(h_gat, wfus[D:2 * D], preferred_element_type=jnp.float32)
        + jnp.dot(h_sage, wfus[2 * D:3 * D], preferred_element_type=jnp.float32)
        + bf_ref[...])


def _tc_post(cnt_parts, asum_parts, s_sage, s_gcn, s_gat, xsr, xw, xg,
             a_src, a_dst, W_sage_l, W_fus, b_gcn, b_gat, b_sage_l, b_fus):
    return pl.pallas_call(
        _post_body,
        grid=(GRID,),
        in_specs=[
            pl.BlockSpec((NW, GB), lambda i: (0, i)),
            pl.BlockSpec((NW, GB), lambda i: (0, i)),
            pl.BlockSpec((NC, GB, D), lambda i: (0, i, 0)),
            pl.BlockSpec((NC, GB, D), lambda i: (0, i, 0)),
            pl.BlockSpec((NC, GB, D), lambda i: (0, i, 0)),
            pl.BlockSpec((GB, D), lambda i: (i, 0)),
            pl.BlockSpec((GB, D), lambda i: (i, 0)),
            pl.BlockSpec((GB, D), lambda i: (i, 0)),
            pl.BlockSpec((GB, 1), lambda i: (i, 0)),
            pl.BlockSpec((GB, 1), lambda i: (i, 0)),
            pl.BlockSpec((D, D), lambda i: (0, 0)),
            pl.BlockSpec((3 * D, D), lambda i: (0, 0)),
            pl.BlockSpec((1, D), lambda i: (0, 0)),
            pl.BlockSpec((1, D), lambda i: (0, 0)),
            pl.BlockSpec((1, D), lambda i: (0, 0)),
            pl.BlockSpec((1, D), lambda i: (0, 0)),
        ],
        out_specs=pl.BlockSpec((GB, D), lambda i: (i, 0)),
        out_shape=jax.ShapeDtypeStruct((NP, D), jnp.float32),
    )(cnt_parts, asum_parts, s_sage, s_gcn, s_gat, xsr, xw, xg, a_src, a_dst,
      W_sage_l, W_fus,
      b_gcn.reshape(1, D), b_gat.reshape(1, D), b_sage_l.reshape(1, D),
      b_fus.reshape(1, D))


def _sc0_body(row_hbm, col_hbm, asrc_hbm, adst_hbm,
              ae_hbm, asum_hbm, cnt_hbm,
              asrc_v, adst_v, row_v, col_v, ae_v, asum_v, cnt_v):
    c = lax.axis_index("c")
    s = lax.axis_index("s")
    wid = s * NC + c
    base = wid * EPT

    pltpu.sync_copy(asrc_hbm, asrc_v)
    pltpu.sync_copy(adst_hbm, adst_v)
    pltpu.sync_copy(row_hbm.at[pl.ds(base, EPT)], row_v)
    pltpu.sync_copy(col_hbm.at[pl.ds(base, EPT)], col_v)

    zero16 = jnp.zeros((L,), jnp.float32)

    @pl.loop(0, NP, step=L)
    def _(i):
        asum_v[pl.ds(i, L)] = zero16
        cnt_v[pl.ds(i, L)] = zero16

    ones = jnp.ones((L,), jnp.float32)

    @pl.loop(0, EPT, step=L)
    def _(i):
        r = row_v[pl.ds(i, L)]
        cc = col_v[pl.ds(i, L)]
        sa = plsc.load_gather(asrc_v, [r])
        da = plsc.load_gather(adst_v, [cc])
        al = sa + da
        ae = jnp.exp(jnp.maximum(al, 0.2 * al))
        ae_v[pl.ds(i, L)] = ae
        plsc.addupdate_scatter(asum_v, [cc], ae)
        plsc.addupdate_scatter(cnt_v, [cc], ones)

    pltpu.sync_copy(ae_v, ae_hbm.at[pl.ds(base, EPT)])
    pltpu.sync_copy(asum_v, asum_hbm.at[wid])
    pltpu.sync_copy(cnt_v, cnt_hbm.at[wid])


def _sc_edge_scalars(row, col, a_src, a_dst):
    kern = pl.kernel(
        _sc0_body,
        out_type=(jax.ShapeDtypeStruct((E_PAD,), jnp.float32),
                  jax.ShapeDtypeStruct((NW, NP), jnp.float32),
                  jax.ShapeDtypeStruct((NW, NP), jnp.float32)),
        mesh=_mesh,
        scratch_types=[
            pltpu.VMEM((NP,), jnp.float32),
            pltpu.VMEM((NP,), jnp.float32),
            pltpu.VMEM((EPT,), jnp.int32),
            pltpu.VMEM((EPT,), jnp.int32),
            pltpu.VMEM((EPT,), jnp.float32),
            pltpu.VMEM((NP,), jnp.float32),
            pltpu.VMEM((NP,), jnp.float32),
        ],
        compiler_params=_sc_params,
    )
    return kern(row, col, a_src, a_dst)


FBLK = 64
FNSTEP = EPT // FBLK
NBUF = 4
NRING = 8
DRB = 64


def _make_agg_body(spec):
    n = len(spec)
    scaled_any = any(spec)

    def body(*refs):
        tabs = refs[0:n]
        row_hbm, col_hbm = refs[n:n + 2]
        k = n + 2
        if scaled_any:
            ae_hbm = refs[k]
            k += 1
        outs = refs[k:k + n]
        k += n
        acc_sh, rowr, colr = refs[k:k + 3]
        k += 3
        if scaled_any:
            aer = refs[k]
            k += 1
        bufs = refs[k:k + NBUF]
        k += NBUF
        zb = refs[k]
        k += 1
        isems = refs[k:k + NRING]
        k += NRING
        gsems = refs[k:k + NBUF]
        k += NBUF
        ssems = refs[k:k + NBUF]

        c = lax.axis_index("c")
        s = lax.axis_index("s")
        wid = s * NC + c
        sbase = wid * FNSTEP
        rbase = s * RPT
        tabx_hbm = tabs[0]

        zero16 = jnp.zeros((L,), jnp.float32)

        def start_idx(step, slot, scaled):
            pltpu.async_copy(row_hbm.at[sbase + step], rowr.at[slot], isems[slot])
            pltpu.async_copy(col_hbm.at[sbase + step], colr.at[slot], isems[slot])
            if scaled:
                pltpu.async_copy(ae_hbm.at[sbase + step], aer.at[slot],
                                 isems[slot])

        def wait_idx(slot, scaled):
            pltpu.make_async_copy(row_hbm.at[0], rowr.at[slot], isems[slot]).wait()
            pltpu.make_async_copy(col_hbm.at[0], colr.at[slot], isems[slot]).wait()
            if scaled:
                pltpu.make_async_copy(ae_hbm.at[0], aer.at[slot],
                                      isems[slot]).wait()

        def start_gather(tab_hbm, slot, pb):
            pltpu.async_copy(tab_hbm.at[rowr.at[slot]], bufs[pb], gsems[pb])

        def wait_gather(pb):
            pltpu.make_async_copy(tabx_hbm.at[pl.ds(0, FBLK)], bufs[pb],
                                  gsems[pb]).wait()

        def wait_scatter(pb):
            pltpu.make_async_copy(tabx_hbm.at[pl.ds(0, FBLK)], bufs[pb],
                                  ssems[pb]).wait()

        def scale_rows(pb, slot):
            @pl.loop(0, FBLK, unroll=2)
            def _(j):
                a = plsc.load_gather(aer.at[slot],
                                     [jnp.full((L,), j, jnp.int32)])
                for k in range(D // L):
                    sl = pl.ds(k * L, L)
                    bufs[pb][j, sl] = bufs[pb][j, sl] * a

        def zero_acc():
            @pl.loop(0, DRB)
            def _(j):
                for k in range(D // L):
                    zb[j, pl.ds(k * L, L)] = zero16

            @pl.loop(0, RPT, step=DRB)
            def _(r0):
                pltpu.sync_copy(zb, acc_sh.at[pl.ds(rbase + r0, DRB)])

        def prime(tab_hbm, scaled):
            for k in range(NBUF):
                start_idx(k, k, scaled)
            for k in range(NBUF - 1):
                wait_idx(k, scaled)
                start_gather(tab_hbm, k, k)

        def sections(tab_hbm, scaled):
            @pl.loop(0, FNSTEP, step=NRING)
            def _(g):
                for b in range(NRING):
                    gi = g + b
                    s8 = b
                    b4 = b % NBUF
                    wait_gather(b4)
                    if scaled:
                        scale_rows(b4, s8)
                    pltpu.async_copy(bufs[b4], acc_sh.at[colr.at[s8]],
                                     ssems[b4], add=True)

                    @pl.when(gi + NBUF < FNSTEP)
                    def _():
                        start_idx(gi + NBUF, (b + NBUF) % NRING, scaled)

                    @pl.when(gi + 3 < FNSTEP)
                    def _():
                        @pl.when(gi >= 1)
                        def _():
                            wait_scatter((b + 3) % NBUF)

                        wait_idx((b + 3) % NRING, scaled)
                        start_gather(tab_hbm, (b + 3) % NRING, (b + 3) % NBUF)

            for k in range(NBUF):
                wait_scatter(k)

        def drain_to(out_hbm):
            @pl.loop(0, RPT, step=DRB)
            def _(r0):
                pltpu.sync_copy(acc_sh.at[pl.ds(rbase + r0, DRB)], zb)
                pltpu.sync_copy(zb, out_hbm.at[c].at[pl.ds(rbase + r0, DRB)])

        for tab, out, scaled in zip(tabs, outs, spec):
            zero_acc()
            prime(tab, scaled)
            plsc.subcore_barrier()
            sections(tab, scaled)
            plsc.subcore_barrier()
            drain_to(out)

    return body


def _make_agg(spec):
    ot = jax.ShapeDtypeStruct((NC, NP, D), jnp.float32)
    scaled_any = any(spec)
    scratch = [
        pltpu.VMEM_SHARED((NP, D), jnp.float32),
        pltpu.VMEM((NRING, FBLK), jnp.int32),
        pltpu.VMEM((NRING, FBLK), jnp.int32),
    ]
    if scaled_any:
        scratch.append(pltpu.VMEM((NRING, FBLK), jnp.float32))
    scratch += [pltpu.VMEM((FBLK, D), jnp.float32)] * NBUF
    scratch += [pltpu.VMEM((DRB, D), jnp.float32)]
    scratch += [pltpu.SemaphoreType.DMA] * (NRING + 2 * NBUF)
    out_type = ot if len(spec) == 1 else tuple(ot for _ in spec)
    return pl.kernel(
        _make_agg_body(spec),
        out_type=out_type,
        mesh=_mesh,
        scratch_types=scratch,
        compiler_params=_sc_params,
    )


_sc_agg_sage = _make_agg((False,))
_sc_agg_gg = _make_agg((False, True))


def kernel(x, edge_index, W_gcn, b_gcn, W_gat, att_src, att_dst, b_gat,
           W_sage_l, b_sage_l, W_sage_r, W_fus, b_fus):
    row = edge_index[0]
    col = edge_index[1]
    npad = E_PAD - row.shape[0]
    ar = jnp.arange(npad, dtype=jnp.int32)
    row_p = jnp.concatenate([row, (ar * 37) % N])
    col_p = jnp.concatenate([col, N + ar % (NP - N)])

    x_p = jnp.zeros((NP, D), jnp.float32).at[:N].set(x)
    av_s = (W_gat @ att_src).reshape(D, 1)
    av_d = (W_gat @ att_dst).reshape(D, 1)

    xw, xg, asrc, adst, xsr = _tc_pre(x_p, W_gcn, W_gat, av_s, av_d, W_sage_r)

    row2 = row_p.reshape(NW * FNSTEP, FBLK)
    col2 = col_p.reshape(NW * FNSTEP, FBLK)
    ae, asum_parts, cnt_parts = _sc_edge_scalars(
        row_p, col_p, asrc.reshape(NP), adst.reshape(NP))
    ae2 = ae.reshape(NW * FNSTEP, FBLK)
    s_sage = _sc_agg_sage(x_p, row2, col2)
    u = _tc_mid(cnt_parts, xw)
    s_gcn, s_gat = _sc_agg_gg(u, xg, row2, col2, ae2)

    out = _tc_post(cnt_parts, asum_parts, s_sage, s_gcn, s_gat, xsr, xw, xg,
                   asrc, adst, W_sage_l, W_fus, b_gcn, b_gat, b_sage_l, b_fus)
    return out[:N]


# speedup vs baseline: 1.0089x; 1.0013x over previous
import dataclasses

import jax
import jax.numpy as jnp
from jax import lax
from jax.experimental import pallas as pl
from jax.experimental.pallas import tpu as pltpu
from jax.experimental.pallas import tpu_sc as plsc

N = 10000
NP = 10240
D = 128
NC = 2
NS = 16
L = 16
NW = NC * NS
EPT = 10240
E_PAD = NW * EPT
BLK = 128
RPT = NP // NS
GB = 1024
GRID = NP // GB

_mesh = plsc.VectorSubcoreMesh(core_axis_name="c", subcore_axis_name="s")

_sc_params = pltpu.CompilerParams()
if "needs_layout_passes" in pltpu.CompilerParams.__dataclass_fields__:
    _sc_params = dataclasses.replace(_sc_params, needs_layout_passes=False)


def _pre_body(x_ref, wg_ref, wt_ref, att2_ref, wsr_ref,
              xw_o, xg_o, a2_o, xsr_o):
    xb = x_ref[...]
    xw_o[...] = jnp.dot(xb, wg_ref[...], preferred_element_type=jnp.float32)
    xg_o[...] = jnp.dot(xb, wt_ref[...], preferred_element_type=jnp.float32)
    a2_o[...] = jnp.dot(xb, att2_ref[...], preferred_element_type=jnp.float32)
    xsr_o[...] = jnp.dot(xb, wsr_ref[...], preferred_element_type=jnp.float32)


def _tc_pre(x, W_gcn, W_gat, att2, W_sage_r):
    mat = jax.ShapeDtypeStruct((NP, D), jnp.float32)
    return pl.pallas_call(
        _pre_body,
        grid=(GRID,),
        in_specs=[pl.BlockSpec((GB, D), lambda i: (i, 0)),
                  pl.BlockSpec((D, D), lambda i: (0, 0)),
                  pl.BlockSpec((D, D), lambda i: (0, 0)),
                  pl.BlockSpec((D, D), lambda i: (0, 0)),
                  pl.BlockSpec((D, D), lambda i: (0, 0))],
        out_specs=[pl.BlockSpec((GB, D), lambda i: (i, 0)),
                   pl.BlockSpec((GB, D), lambda i: (i, 0)),
                   pl.BlockSpec((GB, D), lambda i: (i, 0)),
                   pl.BlockSpec((GB, D), lambda i: (i, 0))],
        out_shape=(mat, mat, mat, mat),
    )(x, W_gcn, W_gat, att2, W_sage_r)


def _mid_body(cntp_ref, xw_ref, u_ref):
    cnt = jnp.sum(cntp_ref[...], axis=0)
    dinv = lax.rsqrt(cnt + 1.0)
    u_ref[...] = dinv[:, None] * xw_ref[...]


def _tc_mid(cnt_parts, xw):
    return pl.pallas_call(
        _mid_body,
        grid=(GRID,),
        in_specs=[pl.BlockSpec((NW, GB), lambda i: (0, i)),
                  pl.BlockSpec((GB, D), lambda i: (i, 0))],
        out_specs=pl.BlockSpec((GB, D), lambda i: (i, 0)),
        out_shape=jax.ShapeDtypeStruct((NP, D), jnp.float32),
    )(cnt_parts, xw)


def _post_body(cntp_ref, asump_ref, ssage_ref, sgcn_ref, sgat_ref, xsr_ref,
               xw_ref, xg_ref, as_ref, ad_ref, wsl_ref, wfus_ref, bg_ref,
               bga_ref, bsl_ref, bf_ref, o_ref):
    cnt = jnp.sum(cntp_ref[...], axis=0)
    asum_e = jnp.sum(asump_ref[...], axis=0)
    s_sage = ssage_ref[0] + ssage_ref[1]
    s_gcn = sgcn_ref[0] + sgcn_ref[1]
    s_gat = sgat_ref[0] + sgat_ref[1]
    xw = xw_ref[...]
    xg = xg_ref[...]

    dinv = lax.rsqrt(cnt + 1.0)[:, None]
    h_gcn = jnp.maximum(dinv * s_gcn + dinv * dinv * xw + bg_ref[...], 0.0)

    al = as_ref[...] + ad_ref[...]
    ae_self = jnp.exp(jnp.maximum(al, 0.2 * al))
    denom = asum_e[:, None] + ae_self + 1e-16
    h_gat = jnp.maximum((s_gat + ae_self * xg) / denom + bga_ref[...], 0.0)

    mean = s_sage / jnp.maximum(cnt, 1.0)[:, None]
    h_sage = jnp.maximum(
        jnp.dot(mean, wsl_ref[...], preferred_element_type=jnp.float32)
        + bsl_ref[...] + xsr_ref[...],
        0.0)

    wfus = wfus_ref[...]
    o_ref[...] = (
        jnp.dot(h_gcn, wfus[0:D], preferred_element_type=jnp.float32)
        + jnp.dot(h_gat, wfus[D:2 * D], preferred_element_type=jnp.float32)
        + jnp.dot(h_sage, wfus[2 * D:3 * D], preferred_element_type=jnp.float32)
        + bf_ref[...])


def _tc_post(cnt_parts, asum_parts, s_sage, s_gcn, s_gat, xsr, xw, xg,
             a_src, a_dst, W_sage_l, W_fus, b_gcn, b_gat, b_sage_l, b_fus):
    return pl.pallas_call(
        _post_body,
        grid=(GRID,),
        in_specs=[
            pl.BlockSpec((NW, GB), lambda i: (0, i)),
            pl.BlockSpec((NW, GB), lambda i: (0, i)),
            pl.BlockSpec((NC, GB, D), lambda i: (0, i, 0)),
            pl.BlockSpec((NC, GB, D), lambda i: (0, i, 0)),
            pl.BlockSpec((NC, GB, D), lambda i: (0, i, 0)),
            pl.BlockSpec((GB, D), lambda i: (i, 0)),
            pl.BlockSpec((GB, D), lambda i: (i, 0)),
            pl.BlockSpec((GB, D), lambda i: (i, 0)),
            pl.BlockSpec((GB, 1), lambda i: (i, 0)),
            pl.BlockSpec((GB, 1), lambda i: (i, 0)),
            pl.BlockSpec((D, D), lambda i: (0, 0)),
            pl.BlockSpec((3 * D, D), lambda i: (0, 0)),
            pl.BlockSpec((1, D), lambda i: (0, 0)),
            pl.BlockSpec((1, D), lambda i: (0, 0)),
            pl.BlockSpec((1, D), lambda i: (0, 0)),
            pl.BlockSpec((1, D), lambda i: (0, 0)),
        ],
        out_specs=pl.BlockSpec((GB, D), lambda i: (i, 0)),
        out_shape=jax.ShapeDtypeStruct((NP, D), jnp.float32),
    )(cnt_parts, asum_parts, s_sage, s_gcn, s_gat, xsr, xw, xg, a_src, a_dst,
      W_sage_l, W_fus,
      b_gcn.reshape(1, D), b_gat.reshape(1, D), b_sage_l.reshape(1, D),
      b_fus.reshape(1, D))


def _sc0_body(row_hbm, col_hbm, asrc_hbm, adst_hbm,
              ae_hbm, asum_hbm, cnt_hbm,
              asrc_v, adst_v, row_v, col_v, ae_v, asum_v, cnt_v):
    c = lax.axis_index("c")
    s = lax.axis_index("s")
    wid = s * NC + c
    base = wid * EPT

    pltpu.sync_copy(asrc_hbm, asrc_v)
    pltpu.sync_copy(adst_hbm, adst_v)
    pltpu.sync_copy(row_hbm.at[pl.ds(base, EPT)], row_v)
    pltpu.sync_copy(col_hbm.at[pl.ds(base, EPT)], col_v)

    zero16 = jnp.zeros((L,), jnp.float32)

    @pl.loop(0, NP, step=L)
    def _(i):
        asum_v[pl.ds(i, L)] = zero16
        cnt_v[pl.ds(i, L)] = zero16

    ones = jnp.ones((L,), jnp.float32)

    @pl.loop(0, EPT, step=L)
    def _(i):
        r = row_v[pl.ds(i, L)]
        cc = col_v[pl.ds(i, L)]
        sa = plsc.load_gather(asrc_v, [r])
        da = plsc.load_gather(adst_v, [cc])
        al = sa + da
        ae = jnp.exp(jnp.maximum(al, 0.2 * al))
        ae_v[pl.ds(i, L)] = ae
        plsc.addupdate_scatter(asum_v, [cc], ae)
        plsc.addupdate_scatter(cnt_v, [cc], ones)

    pltpu.sync_copy(ae_v, ae_hbm.at[pl.ds(base, EPT)])
    pltpu.sync_copy(asum_v, asum_hbm.at[wid])
    pltpu.sync_copy(cnt_v, cnt_hbm.at[wid])


def _sc_edge_scalars(row, col, a_src, a_dst):
    kern = pl.kernel(
        _sc0_body,
        out_type=(jax.ShapeDtypeStruct((E_PAD,), jnp.float32),
                  jax.ShapeDtypeStruct((NW, NP), jnp.float32),
                  jax.ShapeDtypeStruct((NW, NP), jnp.float32)),
        mesh=_mesh,
        scratch_types=[
            pltpu.VMEM((NP,), jnp.float32),
            pltpu.VMEM((NP,), jnp.float32),
            pltpu.VMEM((EPT,), jnp.int32),
            pltpu.VMEM((EPT,), jnp.int32),
            pltpu.VMEM((EPT,), jnp.float32),
            pltpu.VMEM((NP,), jnp.float32),
            pltpu.VMEM((NP,), jnp.float32),
        ],
        compiler_params=_sc_params,
    )
    return kern(row, col, a_src, a_dst)


FBLK = 64
FNSTEP = EPT // FBLK
NBUF = 4
NRING = 8
DRB = 64


def _make_agg_body(spec):
    n = len(spec)
    scaled_any = any(spec)

    def body(*refs):
        tabs = refs[0:n]
        row_hbm, col_hbm = refs[n:n + 2]
        k = n + 2
        if scaled_any:
            ae_hbm = refs[k]
            k += 1
        outs = refs[k:k + n]
        k += n
        acc_sh, rowr, colr = refs[k:k + 3]
        k += 3
        if scaled_any:
            aer = refs[k]
            k += 1
        bufs = refs[k:k + NBUF]
        k += NBUF
        zb = refs[k]
        k += 1
        isems = refs[k:k + NRING]
        k += NRING
        gsems = refs[k:k + NBUF]
        k += NBUF
        ssems = refs[k:k + NBUF]

        c = lax.axis_index("c")
        s = lax.axis_index("s")
        wid = s * NC + c
        sbase = wid * FNSTEP
        rbase = s * RPT
        tabx_hbm = tabs[0]

        zero16 = jnp.zeros((L,), jnp.float32)

        def start_idx(step, slot, scaled):
            pltpu.async_copy(row_hbm.at[sbase + step], rowr.at[slot], isems[slot])
            pltpu.async_copy(col_hbm.at[sbase + step], colr.at[slot], isems[slot])
            if scaled:
                pltpu.async_copy(ae_hbm.at[sbase + step], aer.at[slot],
                                 isems[slot])

        def wait_idx(slot, scaled):
            pltpu.make_async_copy(row_hbm.at[0], rowr.at[slot], isems[slot]).wait()
            pltpu.make_async_copy(col_hbm.at[0], colr.at[slot], isems[slot]).wait()
            if scaled:
                pltpu.make_async_copy(ae_hbm.at[0], aer.at[slot],
                                      isems[slot]).wait()

        def start_gather(tab_hbm, slot, pb):
            pltpu.async_copy(tab_hbm.at[rowr.at[slot]], bufs[pb], gsems[pb])

        def wait_gather(pb):
            pltpu.make_async_copy(tabx_hbm.at[pl.ds(0, FBLK)], bufs[pb],
                                  gsems[pb]).wait()

        def wait_scatter(pb):
            pltpu.make_async_copy(tabx_hbm.at[pl.ds(0, FBLK)], bufs[pb],
                                  ssems[pb]).wait()

        def scale_rows(pb, slot):
            @pl.loop(0, FBLK, unroll=4)
            def _(j):
                a = plsc.load_gather(aer.at[slot],
                                     [jnp.full((L,), j, jnp.int32)])
                for k in range(D // L):
                    sl = pl.ds(k * L, L)
                    bufs[pb][j, sl] = bufs[pb][j, sl] * a

        def zero_acc():
            @pl.loop(0, DRB)
            def _(j):
                for k in range(D // L):
                    zb[j, pl.ds(k * L, L)] = zero16

            @pl.loop(0, RPT, step=DRB)
            def _(r0):
                pltpu.sync_copy(zb, acc_sh.at[pl.ds(rbase + r0, DRB)])

        def prime(tab_hbm, scaled):
            for k in range(NBUF):
                start_idx(k, k, scaled)
            for k in range(NBUF - 1):
                wait_idx(k, scaled)
                start_gather(tab_hbm, k, k)

        def sections(tab_hbm, scaled):
            @pl.loop(0, FNSTEP, step=NRING)
            def _(g):
                for b in range(NRING):
                    gi = g + b
                    s8 = b
                    b4 = b % NBUF
                    wait_gather(b4)
                    if scaled:
                        scale_rows(b4, s8)
                    pltpu.async_copy(bufs[b4], acc_sh.at[colr.at[s8]],
                                     ssems[b4], add=True)

                    @pl.when(gi + NBUF < FNSTEP)
                    def _():
                        start_idx(gi + NBUF, (b + NBUF) % NRING, scaled)

                    @pl.when(gi + 3 < FNSTEP)
                    def _():
                        @pl.when(gi >= 1)
                        def _():
                            wait_scatter((b + 3) % NBUF)

                        wait_idx((b + 3) % NRING, scaled)
                        start_gather(tab_hbm, (b + 3) % NRING, (b + 3) % NBUF)

            for k in range(NBUF):
                wait_scatter(k)

        def drain_to(out_hbm):
            @pl.loop(0, RPT, step=DRB)
            def _(r0):
                pltpu.sync_copy(acc_sh.at[pl.ds(rbase + r0, DRB)], zb)
                pltpu.sync_copy(zb, out_hbm.at[c].at[pl.ds(rbase + r0, DRB)])

        for tab, out, scaled in zip(tabs, outs, spec):
            zero_acc()
            prime(tab, scaled)
            plsc.subcore_barrier()
            sections(tab, scaled)
            plsc.subcore_barrier()
            drain_to(out)

    return body


def _make_agg(spec):
    ot = jax.ShapeDtypeStruct((NC, NP, D), jnp.float32)
    scaled_any = any(spec)
    scratch = [
        pltpu.VMEM_SHARED((NP, D), jnp.float32),
        pltpu.VMEM((NRING, FBLK), jnp.int32),
        pltpu.VMEM((NRING, FBLK), jnp.int32),
    ]
    if scaled_any:
        scratch.append(pltpu.VMEM((NRING, FBLK), jnp.float32))
    scratch += [pltpu.VMEM((FBLK, D), jnp.float32)] * NBUF
    scratch += [pltpu.VMEM((DRB, D), jnp.float32)]
    scratch += [pltpu.SemaphoreType.DMA] * (NRING + 2 * NBUF)
    out_type = ot if len(spec) == 1 else tuple(ot for _ in spec)
    return pl.kernel(
        _make_agg_body(spec),
        out_type=out_type,
        mesh=_mesh,
        scratch_types=scratch,
        compiler_params=_sc_params,
    )


_sc_agg_sage = _make_agg((False,))
_sc_agg_gg = _make_agg((False, True))


def kernel(x, edge_index, W_gcn, b_gcn, W_gat, att_src, att_dst, b_gat,
           W_sage_l, b_sage_l, W_sage_r, W_fus, b_fus):
    row = edge_index[0]
    col = edge_index[1]
    npad = E_PAD - row.shape[0]
    ar = jnp.arange(npad, dtype=jnp.int32)
    row_p = jnp.concatenate([row, (ar * 37) % N])
    col_p = jnp.concatenate([col, N + ar % (NP - N)])

    x_p = jnp.zeros((NP, D), jnp.float32).at[:N].set(x)
    att2 = (jnp.zeros((D, D), jnp.float32)
            .at[:, 0].set(W_gat @ att_src).at[:, 1].set(W_gat @ att_dst))

    xw, xg, a2, xsr = _tc_pre(x_p, W_gcn, W_gat, att2, W_sage_r)
    asrc = a2[:, 0:1]
    adst = a2[:, 1:2]

    row2 = row_p.reshape(NW * FNSTEP, FBLK)
    col2 = col_p.reshape(NW * FNSTEP, FBLK)
    ae, asum_parts, cnt_parts = _sc_edge_scalars(
        row_p, col_p, asrc.reshape(NP), adst.reshape(NP))
    ae2 = ae.reshape(NW * FNSTEP, FBLK)
    s_sage = _sc_agg_sage(x_p, row2, col2)
    u = _tc_mid(cnt_parts, xw)
    s_gcn, s_gat = _sc_agg_gg(u, xg, row2, col2, ae2)

    out = _tc_post(cnt_parts, asum_parts, s_sage, s_gcn, s_gat, xsr, xw, xg,
                   asrc, adst, W_sage_l, W_fus, b_gcn, b_gat, b_sage_l, b_fus)
    return out[:N]


# speedup vs baseline: 1.0201x; 1.0111x over previous
import dataclasses

import jax
import jax.numpy as jnp
from jax import lax
from jax.experimental import pallas as pl
from jax.experimental.pallas import tpu as pltpu
from jax.experimental.pallas import tpu_sc as plsc

N = 10000
NP = 10240
D = 128
NC = 2
NS = 16
L = 16
NW = NC * NS
EPT = 10240
E_PAD = NW * EPT
BLK = 128
RPT = NP // NS
GB = 1024
GRID = NP // GB

_mesh = plsc.VectorSubcoreMesh(core_axis_name="c", subcore_axis_name="s")

_sc_params = pltpu.CompilerParams()
if "needs_layout_passes" in pltpu.CompilerParams.__dataclass_fields__:
    _sc_params = dataclasses.replace(_sc_params, needs_layout_passes=False)


def _pre_body(x_ref, wg_ref, wt_ref, att2_ref, wsr_ref,
              xw_o, xg_o, a2_o, xsr_o):
    xb = x_ref[...]
    xw_o[...] = jnp.dot(xb, wg_ref[...], preferred_element_type=jnp.float32)
    xg_o[...] = jnp.dot(xb, wt_ref[...], preferred_element_type=jnp.float32)
    a2_o[...] = jnp.dot(xb, att2_ref[...], preferred_element_type=jnp.float32,
                        precision=lax.Precision.HIGHEST)
    xsr_o[...] = jnp.dot(xb, wsr_ref[...], preferred_element_type=jnp.float32)


def _tc_pre(x, W_gcn, W_gat, att2, W_sage_r):
    mat = jax.ShapeDtypeStruct((NP, D), jnp.float32)
    return pl.pallas_call(
        _pre_body,
        grid=(GRID,),
        in_specs=[pl.BlockSpec((GB, D), lambda i: (i, 0)),
                  pl.BlockSpec((D, D), lambda i: (0, 0)),
                  pl.BlockSpec((D, D), lambda i: (0, 0)),
                  pl.BlockSpec((D, 2), lambda i: (0, 0)),
                  pl.BlockSpec((D, D), lambda i: (0, 0))],
        out_specs=[pl.BlockSpec((GB, D), lambda i: (i, 0)),
                   pl.BlockSpec((GB, D), lambda i: (i, 0)),
                   pl.BlockSpec((GB, 2), lambda i: (i, 0)),
                   pl.BlockSpec((GB, D), lambda i: (i, 0))],
        out_shape=(mat, mat, jax.ShapeDtypeStruct((NP, 2), jnp.float32), mat),
    )(x, W_gcn, W_gat, att2, W_sage_r)


def _mid_body(cntp_ref, xw_ref, u_ref):
    cnt = jnp.sum(cntp_ref[...], axis=0)
    dinv = lax.rsqrt(cnt + 1.0)
    u_ref[...] = dinv[:, None] * xw_ref[...]


def _tc_mid(cnt_parts, xw):
    return pl.pallas_call(
        _mid_body,
        grid=(GRID,),
        in_specs=[pl.BlockSpec((NW, GB), lambda i: (0, i)),
                  pl.BlockSpec((GB, D), lambda i: (i, 0))],
        out_specs=pl.BlockSpec((GB, D), lambda i: (i, 0)),
        out_shape=jax.ShapeDtypeStruct((NP, D), jnp.float32),
    )(cnt_parts, xw)


def _post_body(cntp_ref, asump_ref, ssage_ref, sgcn_ref, sgat_ref, xsr_ref,
               xw_ref, xg_ref, a2_ref, wsl_ref, wfus_ref, bg_ref,
               bga_ref, bsl_ref, bf_ref, o_ref):
    cnt = jnp.sum(cntp_ref[...], axis=0)
    asum_e = jnp.sum(asump_ref[...], axis=0)
    s_sage = ssage_ref[0] + ssage_ref[1]
    s_gcn = sgcn_ref[0] + sgcn_ref[1]
    s_gat = sgat_ref[0] + sgat_ref[1]
    xw = xw_ref[...]
    xg = xg_ref[...]

    dinv = lax.rsqrt(cnt + 1.0)[:, None]
    h_gcn = jnp.maximum(dinv * s_gcn + dinv * dinv * xw + bg_ref[...], 0.0)

    a2 = a2_ref[...]
    al = a2[:, 0:1] + a2[:, 1:2]
    ae_self = jnp.exp(jnp.maximum(al, 0.2 * al))
    denom = asum_e[:, None] + ae_self + 1e-16
    h_gat = jnp.maximum((s_gat + ae_self * xg) / denom + bga_ref[...], 0.0)

    mean = s_sage / jnp.maximum(cnt, 1.0)[:, None]
    h_sage = jnp.maximum(
        jnp.dot(mean, wsl_ref[...], preferred_element_type=jnp.float32)
        + bsl_ref[...] + xsr_ref[...],
        0.0)

    wfus = wfus_ref[...]
    o_ref[...] = (
        jnp.dot(h_gcn, wfus[0:D], preferred_element_type=jnp.float32)
        + jnp.dot(h_gat, wfus[D:2 * D], preferred_element_type=jnp.float32)
        + jnp.dot(h_sage, wfus[2 * D:3 * D], preferred_element_type=jnp.float32)
        + bf_ref[...])


def _tc_post(cnt_parts, asum_parts, s_sage, s_gcn, s_gat, xsr, xw, xg,
             a2, W_sage_l, W_fus, b_gcn, b_gat, b_sage_l, b_fus):
    return pl.pallas_call(
        _post_body,
        grid=(GRID,),
        in_specs=[
            pl.BlockSpec((NW, GB), lambda i: (0, i)),
            pl.BlockSpec((NW, GB), lambda i: (0, i)),
            pl.BlockSpec((NC, GB, D), lambda i: (0, i, 0)),
            pl.BlockSpec((NC, GB, D), lambda i: (0, i, 0)),
            pl.BlockSpec((NC, GB, D), lambda i: (0, i, 0)),
            pl.BlockSpec((GB, D), lambda i: (i, 0)),
            pl.BlockSpec((GB, D), lambda i: (i, 0)),
            pl.BlockSpec((GB, D), lambda i: (i, 0)),
            pl.BlockSpec((GB, 2), lambda i: (i, 0)),
            pl.BlockSpec((D, D), lambda i: (0, 0)),
            pl.BlockSpec((3 * D, D), lambda i: (0, 0)),
            pl.BlockSpec((1, D), lambda i: (0, 0)),
            pl.BlockSpec((1, D), lambda i: (0, 0)),
            pl.BlockSpec((1, D), lambda i: (0, 0)),
            pl.BlockSpec((1, D), lambda i: (0, 0)),
        ],
        out_specs=pl.BlockSpec((GB, D), lambda i: (i, 0)),
        out_shape=jax.ShapeDtypeStruct((NP, D), jnp.float32),
    )(cnt_parts, asum_parts, s_sage, s_gcn, s_gat, xsr, xw, xg, a2,
      W_sage_l, W_fus,
      b_gcn.reshape(1, D), b_gat.reshape(1, D), b_sage_l.reshape(1, D),
      b_fus.reshape(1, D))


def _sc0_body(row_hbm, col_hbm, a2_hbm,
              ae_hbm, asum_hbm, cnt_hbm,
              a2_v, row_v, col_v, ae_v, asum_v, cnt_v):
    c = lax.axis_index("c")
    s = lax.axis_index("s")
    wid = s * NC + c
    base = wid * EPT

    pltpu.sync_copy(a2_hbm, a2_v)
    pltpu.sync_copy(row_hbm.at[pl.ds(base, EPT)], row_v)
    pltpu.sync_copy(col_hbm.at[pl.ds(base, EPT)], col_v)

    zero16 = jnp.zeros((L,), jnp.float32)

    @pl.loop(0, NP, step=L)
    def _(i):
        asum_v[pl.ds(i, L)] = zero16
        cnt_v[pl.ds(i, L)] = zero16

    ones = jnp.ones((L,), jnp.float32)
    ones_i = jnp.ones((L,), jnp.int32)

    @pl.loop(0, EPT, step=L)
    def _(i):
        r = row_v[pl.ds(i, L)]
        cc = col_v[pl.ds(i, L)]
        sa = plsc.load_gather(a2_v, [r + r])
        da = plsc.load_gather(a2_v, [cc + cc + ones_i])
        al = sa + da
        ae = jnp.exp(jnp.maximum(al, 0.2 * al))
        ae_v[pl.ds(i, L)] = ae
        plsc.addupdate_scatter(asum_v, [cc], ae)
        plsc.addupdate_scatter(cnt_v, [cc], ones)

    pltpu.sync_copy(ae_v, ae_hbm.at[pl.ds(base, EPT)])
    pltpu.sync_copy(asum_v, asum_hbm.at[wid])
    pltpu.sync_copy(cnt_v, cnt_hbm.at[wid])


def _sc_edge_scalars(row, col, a2):
    kern = pl.kernel(
        _sc0_body,
        out_type=(jax.ShapeDtypeStruct((E_PAD,), jnp.float32),
                  jax.ShapeDtypeStruct((NW, NP), jnp.float32),
                  jax.ShapeDtypeStruct((NW, NP), jnp.float32)),
        mesh=_mesh,
        scratch_types=[
            pltpu.VMEM((2 * NP,), jnp.float32),
            pltpu.VMEM((EPT,), jnp.int32),
            pltpu.VMEM((EPT,), jnp.int32),
            pltpu.VMEM((EPT,), jnp.float32),
            pltpu.VMEM((NP,), jnp.float32),
            pltpu.VMEM((NP,), jnp.float32),
        ],
        compiler_params=_sc_params,
    )
    return kern(row, col, a2)


FBLK = 64
FNSTEP = EPT // FBLK
NBUF = 4
NRING = 8
DRB = 64


def _make_agg_body(spec):
    n = len(spec)
    scaled_any = any(spec)

    def body(*refs):
        tabs = refs[0:n]
        row_hbm, col_hbm = refs[n:n + 2]
        k = n + 2
        if scaled_any:
            ae_hbm = refs[k]
            k += 1
        outs = refs[k:k + n]
        k += n
        acc_sh, rowr, colr = refs[k:k + 3]
        k += 3
        if scaled_any:
            aer = refs[k]
            k += 1
        bufs = refs[k:k + NBUF]
        k += NBUF
        zb = refs[k]
        k += 1
        isems = refs[k:k + NRING]
        k += NRING
        gsems = refs[k:k + NBUF]
        k += NBUF
        ssems = refs[k:k + NBUF]

        c = lax.axis_index("c")
        s = lax.axis_index("s")
        wid = s * NC + c
        sbase = wid * FNSTEP
        rbase = s * RPT
        tabx_hbm = tabs[0]

        zero16 = jnp.zeros((L,), jnp.float32)

        def start_idx(step, slot, scaled):
            pltpu.async_copy(row_hbm.at[sbase + step], rowr.at[slot], isems[slot])
            pltpu.async_copy(col_hbm.at[sbase + step], colr.at[slot], isems[slot])
            if scaled:
                pltpu.async_copy(ae_hbm.at[sbase + step], aer.at[slot],
                                 isems[slot])

        def wait_idx(slot, scaled):
            pltpu.make_async_copy(row_hbm.at[0], rowr.at[slot], isems[slot]).wait()
            pltpu.make_async_copy(col_hbm.at[0], colr.at[slot], isems[slot]).wait()
            if scaled:
                pltpu.make_async_copy(ae_hbm.at[0], aer.at[slot],
                                      isems[slot]).wait()

        def start_gather(tab_hbm, slot, pb):
            pltpu.async_copy(tab_hbm.at[rowr.at[slot]], bufs[pb], gsems[pb])

        def wait_gather(pb):
            pltpu.make_async_copy(tabx_hbm.at[pl.ds(0, FBLK)], bufs[pb],
                                  gsems[pb]).wait()

        def wait_scatter(pb):
            pltpu.make_async_copy(tabx_hbm.at[pl.ds(0, FBLK)], bufs[pb],
                                  ssems[pb]).wait()

        def scale_rows(pb, slot):
            @pl.loop(0, FBLK, unroll=4)
            def _(j):
                a = plsc.load_gather(aer.at[slot],
                                     [jnp.full((L,), j, jnp.int32)])
                for k in range(D // L):
                    sl = pl.ds(k * L, L)
                    bufs[pb][j, sl] = bufs[pb][j, sl] * a

        def zero_acc():
            @pl.loop(0, DRB)
            def _(j):
                for k in range(D // L):
                    zb[j, pl.ds(k * L, L)] = zero16

            @pl.loop(0, RPT, step=DRB)
            def _(r0):
                pltpu.sync_copy(zb, acc_sh.at[pl.ds(rbase + r0, DRB)])

        def prime(tab_hbm, scaled):
            for k in range(NBUF):
                start_idx(k, k, scaled)
            for k in range(NBUF - 1):
                wait_idx(k, scaled)
                start_gather(tab_hbm, k, k)

        def sections(tab_hbm, scaled):
            @pl.loop(0, FNSTEP, step=NRING)
            def _(g):
                for b in range(NRING):
                    gi = g + b
                    s8 = b
                    b4 = b % NBUF
                    wait_gather(b4)
                    if scaled:
                        scale_rows(b4, s8)
                    pltpu.async_copy(bufs[b4], acc_sh.at[colr.at[s8]],
                                     ssems[b4], add=True)

                    @pl.when(gi + NBUF < FNSTEP)
                    def _():
                        start_idx(gi + NBUF, (b + NBUF) % NRING, scaled)

                    @pl.when(gi + 3 < FNSTEP)
                    def _():
                        @pl.when(gi >= 1)
                        def _():
                            wait_scatter((b + 3) % NBUF)

                        wait_idx((b + 3) % NRING, scaled)
                        start_gather(tab_hbm, (b + 3) % NRING, (b + 3) % NBUF)

            for k in range(NBUF):
                wait_scatter(k)

        def drain_to(out_hbm):
            @pl.loop(0, RPT, step=DRB)
            def _(r0):
                pltpu.sync_copy(acc_sh.at[pl.ds(rbase + r0, DRB)], zb)
                pltpu.sync_copy(zb, out_hbm.at[c].at[pl.ds(rbase + r0, DRB)])

        for tab, out, scaled in zip(tabs, outs, spec):
            zero_acc()
            prime(tab, scaled)
            plsc.subcore_barrier()
            sections(tab, scaled)
            plsc.subcore_barrier()
            drain_to(out)

    return body


def _make_agg(spec):
    ot = jax.ShapeDtypeStruct((NC, NP, D), jnp.float32)
    scaled_any = any(spec)
    scratch = [
        pltpu.VMEM_SHARED((NP, D), jnp.float32),
        pltpu.VMEM((NRING, FBLK), jnp.int32),
        pltpu.VMEM((NRING, FBLK), jnp.int32),
    ]
    if scaled_any:
        scratch.append(pltpu.VMEM((NRING, FBLK), jnp.float32))
    scratch += [pltpu.VMEM((FBLK, D), jnp.float32)] * NBUF
    scratch += [pltpu.VMEM((DRB, D), jnp.float32)]
    scratch += [pltpu.SemaphoreType.DMA] * (NRING + 2 * NBUF)
    out_type = ot if len(spec) == 1 else tuple(ot for _ in spec)
    return pl.kernel(
        _make_agg_body(spec),
        out_type=out_type,
        mesh=_mesh,
        scratch_types=scratch,
        compiler_params=_sc_params,
    )


_sc_agg_sage = _make_agg((False,))
_sc_agg_gg = _make_agg((False, True))


def kernel(x, edge_index, W_gcn, b_gcn, W_gat, att_src, att_dst, b_gat,
           W_sage_l, b_sage_l, W_sage_r, W_fus, b_fus):
    row = edge_index[0]
    col = edge_index[1]
    npad = E_PAD - row.shape[0]
    ar = jnp.arange(npad, dtype=jnp.int32)
    row_p = jnp.concatenate([row, (ar * 37) % N])
    col_p = jnp.concatenate([col, N + ar % (NP - N)])

    x_p = jnp.zeros((NP, D), jnp.float32).at[:N].set(x)
    att2 = jnp.stack([W_gat @ att_src, W_gat @ att_dst], axis=1)

    xw, xg, a2, xsr = _tc_pre(x_p, W_gcn, W_gat, att2, W_sage_r)

    row2 = row_p.reshape(NW * FNSTEP, FBLK)
    col2 = col_p.reshape(NW * FNSTEP, FBLK)
    ae, asum_parts, cnt_parts = _sc_edge_scalars(row_p, col_p,
                                                 a2.reshape(2 * NP))
    ae2 = ae.reshape(NW * FNSTEP, FBLK)
    s_sage = _sc_agg_sage(x_p, row2, col2)
    u = _tc_mid(cnt_parts, xw)
    s_gcn, s_gat = _sc_agg_gg(u, xg, row2, col2, ae2)

    out = _tc_post(cnt_parts, asum_parts, s_sage, s_gcn, s_gat, xsr, xw, xg,
                   a2, W_sage_l, W_fus, b_gcn, b_gat, b_sage_l, b_fus)
    return out[:N]


# speedup vs baseline: 1.0203x; 1.0002x over previous
import dataclasses

import jax
import jax.numpy as jnp
from jax import lax
from jax.experimental import pallas as pl
from jax.experimental.pallas import tpu as pltpu
from jax.experimental.pallas import tpu_sc as plsc

N = 10000
NP = 10240
D = 128
NC = 2
NS = 16
L = 16
NW = NC * NS
EPT = 10240
E_PAD = NW * EPT
RPT = NP // NS
GB = 1024
GRID = NP // GB

_mesh = plsc.VectorSubcoreMesh(core_axis_name="c", subcore_axis_name="s")

_sc_params = pltpu.CompilerParams()
if "needs_layout_passes" in pltpu.CompilerParams.__dataclass_fields__:
    _sc_params = dataclasses.replace(_sc_params, needs_layout_passes=False)


def _pre_body(x_ref, wg_ref, wt_ref, att2_ref, wsr_ref,
              xw_o, xg_o, a2_o, xsr_o):
    xb = x_ref[...]
    xw_o[...] = jnp.dot(xb, wg_ref[...], preferred_element_type=jnp.float32)
    xg_o[...] = jnp.dot(xb, wt_ref[...], preferred_element_type=jnp.float32)
    a2_o[...] = jnp.dot(xb, att2_ref[...], preferred_element_type=jnp.float32,
                        precision=lax.Precision.HIGHEST)
    xsr_o[...] = jnp.dot(xb, wsr_ref[...], preferred_element_type=jnp.float32)


def _tc_pre(x, W_gcn, W_gat, att2, W_sage_r):
    mat = jax.ShapeDtypeStruct((NP, D), jnp.float32)
    return pl.pallas_call(
        _pre_body,
        grid=(GRID,),
        in_specs=[pl.BlockSpec((GB, D), lambda i: (i, 0)),
                  pl.BlockSpec((D, D), lambda i: (0, 0)),
                  pl.BlockSpec((D, D), lambda i: (0, 0)),
                  pl.BlockSpec((D, 2), lambda i: (0, 0)),
                  pl.BlockSpec((D, D), lambda i: (0, 0))],
        out_specs=[pl.BlockSpec((GB, D), lambda i: (i, 0)),
                   pl.BlockSpec((GB, D), lambda i: (i, 0)),
                   pl.BlockSpec((GB, 2), lambda i: (i, 0)),
                   pl.BlockSpec((GB, D), lambda i: (i, 0))],
        out_shape=(mat, mat, jax.ShapeDtypeStruct((NP, 2), jnp.float32), mat),
    )(x, W_gcn, W_gat, att2, W_sage_r)


def _mid_body(cntp_ref, xw_ref, u_ref):
    cnt = jnp.sum(cntp_ref[...], axis=0)
    dinv = lax.rsqrt(cnt + 1.0)
    u_ref[...] = dinv[:, None] * xw_ref[...]


def _tc_mid(cnt_parts, xw):
    return pl.pallas_call(
        _mid_body,
        grid=(GRID,),
        in_specs=[pl.BlockSpec((NW, GB), lambda i: (0, i)),
                  pl.BlockSpec((GB, D), lambda i: (i, 0))],
        out_specs=pl.BlockSpec((GB, D), lambda i: (i, 0)),
        out_shape=jax.ShapeDtypeStruct((NP, D), jnp.float32),
    )(cnt_parts, xw)


def _post_body(cntp_ref, asump_ref, ssage_ref, sgcn_ref, sgat_ref, xsr_ref,
               xw_ref, xg_ref, a2_ref, wsl_ref, wfus_ref, bg_ref,
               bga_ref, bsl_ref, bf_ref, o_ref):
    cnt = jnp.sum(cntp_ref[...], axis=0)
    asum_e = jnp.sum(asump_ref[...], axis=0)
    s_sage = ssage_ref[0] + ssage_ref[1]
    s_gcn = sgcn_ref[0] + sgcn_ref[1]
    s_gat = sgat_ref[0] + sgat_ref[1]
    xw = xw_ref[...]
    xg = xg_ref[...]

    dinv = lax.rsqrt(cnt + 1.0)[:, None]
    h_gcn = jnp.maximum(dinv * s_gcn + dinv * dinv * xw + bg_ref[...], 0.0)

    a2 = a2_ref[...]
    al = a2[:, 0:1] + a2[:, 1:2]
    ae_self = jnp.exp(jnp.maximum(al, 0.2 * al))
    denom = asum_e[:, None] + ae_self + 1e-16
    h_gat = jnp.maximum((s_gat + ae_self * xg) / denom + bga_ref[...], 0.0)

    mean = s_sage / jnp.maximum(cnt, 1.0)[:, None]
    h_sage = jnp.maximum(
        jnp.dot(mean, wsl_ref[...], preferred_element_type=jnp.float32)
        + bsl_ref[...] + xsr_ref[...],
        0.0)

    wfus = wfus_ref[...]
    o_ref[...] = (
        jnp.dot(h_gcn, wfus[0:D], preferred_element_type=jnp.float32)
        + jnp.dot(h_gat, wfus[D:2 * D], preferred_element_type=jnp.float32)
        + jnp.dot(h_sage, wfus[2 * D:3 * D], preferred_element_type=jnp.float32)
        + bf_ref[...])


def _tc_post(cnt_parts, asum_parts, s_sage, s_gcn, s_gat, xsr, xw, xg,
             a2, W_sage_l, W_fus, b_gcn, b_gat, b_sage_l, b_fus):
    return pl.pallas_call(
        _post_body,
        grid=(GRID,),
        in_specs=[
            pl.BlockSpec((NW, GB), lambda i: (0, i)),
            pl.BlockSpec((NW, GB), lambda i: (0, i)),
            pl.BlockSpec((NC, GB, D), lambda i: (0, i, 0)),
            pl.BlockSpec((NC, GB, D), lambda i: (0, i, 0)),
            pl.BlockSpec((NC, GB, D), lambda i: (0, i, 0)),
            pl.BlockSpec((GB, D), lambda i: (i, 0)),
            pl.BlockSpec((GB, D), lambda i: (i, 0)),
            pl.BlockSpec((GB, D), lambda i: (i, 0)),
            pl.BlockSpec((GB, 2), lambda i: (i, 0)),
            pl.BlockSpec((D, D), lambda i: (0, 0)),
            pl.BlockSpec((3 * D, D), lambda i: (0, 0)),
            pl.BlockSpec((1, D), lambda i: (0, 0)),
            pl.BlockSpec((1, D), lambda i: (0, 0)),
            pl.BlockSpec((1, D), lambda i: (0, 0)),
            pl.BlockSpec((1, D), lambda i: (0, 0)),
        ],
        out_specs=pl.BlockSpec((GB, D), lambda i: (i, 0)),
        out_shape=jax.ShapeDtypeStruct((NP, D), jnp.float32),
    )(cnt_parts, asum_parts, s_sage, s_gcn, s_gat, xsr, xw, xg, a2,
      W_sage_l, W_fus,
      b_gcn.reshape(1, D), b_gat.reshape(1, D), b_sage_l.reshape(1, D),
      b_fus.reshape(1, D))


def _sc0_body(row_hbm, col_hbm, a2_hbm,
              ae_hbm, asum_hbm, cnt_hbm,
              a2_v, row_v, col_v, ae_v, asum_v, cnt_v):
    c = lax.axis_index("c")
    s = lax.axis_index("s")
    wid = s * NC + c
    base = wid * EPT

    pltpu.sync_copy(a2_hbm, a2_v)
    pltpu.sync_copy(row_hbm.at[pl.ds(base, EPT)], row_v)
    pltpu.sync_copy(col_hbm.at[pl.ds(base, EPT)], col_v)

    zero16 = jnp.zeros((L,), jnp.float32)

    @pl.loop(0, NP, step=L)
    def _(i):
        asum_v[pl.ds(i, L)] = zero16
        cnt_v[pl.ds(i, L)] = zero16

    ones = jnp.ones((L,), jnp.float32)
    ones_i = jnp.ones((L,), jnp.int32)

    @pl.loop(0, EPT, step=L)
    def _(i):
        r = row_v[pl.ds(i, L)]
        cc = col_v[pl.ds(i, L)]
        sa = plsc.load_gather(a2_v, [r + r])
        da = plsc.load_gather(a2_v, [cc + cc + ones_i])
        al = sa + da
        ae = jnp.exp(jnp.maximum(al, 0.2 * al))
        ae_v[pl.ds(i, L)] = ae
        plsc.addupdate_scatter(asum_v, [cc], ae)
        plsc.addupdate_scatter(cnt_v, [cc], ones)

    pltpu.sync_copy(ae_v, ae_hbm.at[pl.ds(base, EPT)])
    pltpu.sync_copy(asum_v, asum_hbm.at[wid])
    pltpu.sync_copy(cnt_v, cnt_hbm.at[wid])


def _sc_edge_scalars(row, col, a2):
    kern = pl.kernel(
        _sc0_body,
        out_type=(jax.ShapeDtypeStruct((E_PAD,), jnp.float32),
                  jax.ShapeDtypeStruct((NW, NP), jnp.float32),
                  jax.ShapeDtypeStruct((NW, NP), jnp.float32)),
        mesh=_mesh,
        scratch_types=[
            pltpu.VMEM((2 * NP,), jnp.float32),
            pltpu.VMEM((EPT,), jnp.int32),
            pltpu.VMEM((EPT,), jnp.int32),
            pltpu.VMEM((EPT,), jnp.float32),
            pltpu.VMEM((NP,), jnp.float32),
            pltpu.VMEM((NP,), jnp.float32),
        ],
        compiler_params=_sc_params,
    )
    return kern(row, col, a2)


FBLK = 64
FNSTEP = EPT // FBLK
NBUF = 4
NRING = 8
DRB = 64


def _make_agg_body(spec):
    n = len(spec)
    scaled_any = any(spec)

    def body(*refs):
        tabs = refs[0:n]
        row_hbm, col_hbm = refs[n:n + 2]
        k = n + 2
        if scaled_any:
            ae_hbm = refs[k]
            k += 1
        outs = refs[k:k + n]
        k += n
        acc_sh, rowr, colr = refs[k:k + 3]
        k += 3
        if scaled_any:
            aer = refs[k]
            k += 1
        bufs = refs[k:k + NBUF]
        k += NBUF
        zb = refs[k]
        k += 1
        isems = refs[k:k + NRING]
        k += NRING
        gsems = refs[k:k + NBUF]
        k += NBUF
        ssems = refs[k:k + NBUF]

        c = lax.axis_index("c")
        s = lax.axis_index("s")
        wid = s * NC + c
        sbase = wid * FNSTEP
        rbase = s * RPT
        tabx_hbm = tabs[0]

        zero16 = jnp.zeros((L,), jnp.float32)

        def start_idx(step, slot, scaled):
            pltpu.async_copy(row_hbm.at[sbase + step], rowr.at[slot], isems[slot])
            pltpu.async_copy(col_hbm.at[sbase + step], colr.at[slot], isems[slot])
            if scaled:
                pltpu.async_copy(ae_hbm.at[sbase + step], aer.at[slot],
                                 isems[slot])

        def wait_idx(slot, scaled):
            pltpu.make_async_copy(row_hbm.at[0], rowr.at[slot], isems[slot]).wait()
            pltpu.make_async_copy(col_hbm.at[0], colr.at[slot], isems[slot]).wait()
            if scaled:
                pltpu.make_async_copy(ae_hbm.at[0], aer.at[slot],
                                      isems[slot]).wait()

        def start_gather(tab_hbm, slot, pb):
            pltpu.async_copy(tab_hbm.at[rowr.at[slot]], bufs[pb], gsems[pb])

        def wait_gather(pb):
            pltpu.make_async_copy(tabx_hbm.at[pl.ds(0, FBLK)], bufs[pb],
                                  gsems[pb]).wait()

        def wait_scatter(pb):
            pltpu.make_async_copy(tabx_hbm.at[pl.ds(0, FBLK)], bufs[pb],
                                  ssems[pb]).wait()

        def scale_rows(pb, slot):
            @pl.loop(0, FBLK, unroll=4)
            def _(j):
                a = plsc.load_gather(aer.at[slot],
                                     [jnp.full((L,), j, jnp.int32)])
                for k in range(D // L):
                    sl = pl.ds(k * L, L)
                    bufs[pb][j, sl] = bufs[pb][j, sl] * a

        def zero_acc():
            @pl.loop(0, DRB)
            def _(j):
                for k in range(D // L):
                    zb[j, pl.ds(k * L, L)] = zero16

            @pl.loop(0, RPT, step=DRB)
            def _(r0):
                pltpu.sync_copy(zb, acc_sh.at[pl.ds(rbase + r0, DRB)])

        def prime(tab_hbm, scaled):
            for k in range(NBUF):
                start_idx(k, k, scaled)
            for k in range(NBUF - 1):
                wait_idx(k, scaled)
                start_gather(tab_hbm, k, k)

        def sections(tab_hbm, scaled):
            @pl.loop(0, FNSTEP, step=NRING)
            def _(g):
                for b in range(NRING):
                    gi = g + b
                    s8 = b
                    b4 = b % NBUF
                    wait_gather(b4)
                    if scaled:
                        scale_rows(b4, s8)
                    pltpu.async_copy(bufs[b4], acc_sh.at[colr.at[s8]],
                                     ssems[b4], add=True)

                    @pl.when(gi + NBUF < FNSTEP)
                    def _():
                        start_idx(gi + NBUF, (b + NBUF) % NRING, scaled)

                    @pl.when(gi + 3 < FNSTEP)
                    def _():
                        @pl.when(gi >= 1)
                        def _():
                            wait_scatter((b + 3) % NBUF)

                        wait_idx((b + 3) % NRING, scaled)
                        start_gather(tab_hbm, (b + 3) % NRING, (b + 3) % NBUF)

            for k in range(NBUF):
                wait_scatter(k)

        def drain_to(out_hbm):
            @pl.loop(0, RPT, step=DRB)
            def _(r0):
                pltpu.sync_copy(acc_sh.at[pl.ds(rbase + r0, DRB)], zb)
                pltpu.sync_copy(zb, out_hbm.at[c].at[pl.ds(rbase + r0, DRB)])

        for tab, out, scaled in zip(tabs, outs, spec):
            zero_acc()
            prime(tab, scaled)
            plsc.subcore_barrier()
            sections(tab, scaled)
            plsc.subcore_barrier()
            drain_to(out)

    return body


def _make_agg(spec):
    ot = jax.ShapeDtypeStruct((NC, NP, D), jnp.float32)
    scaled_any = any(spec)
    scratch = [
        pltpu.VMEM_SHARED((NP, D), jnp.float32),
        pltpu.VMEM((NRING, FBLK), jnp.int32),
        pltpu.VMEM((NRING, FBLK), jnp.int32),
    ]
    if scaled_any:
        scratch.append(pltpu.VMEM((NRING, FBLK), jnp.float32))
    scratch += [pltpu.VMEM((FBLK, D), jnp.float32)] * NBUF
    scratch += [pltpu.VMEM((DRB, D), jnp.float32)]
    scratch += [pltpu.SemaphoreType.DMA] * (NRING + 2 * NBUF)
    out_type = ot if len(spec) == 1 else tuple(ot for _ in spec)
    return pl.kernel(
        _make_agg_body(spec),
        out_type=out_type,
        mesh=_mesh,
        scratch_types=scratch,
        compiler_params=_sc_params,
    )


_sc_agg_sage = _make_agg((False,))
_sc_agg_gg = _make_agg((False, True))


def kernel(x, edge_index, W_gcn, b_gcn, W_gat, att_src, att_dst, b_gat,
           W_sage_l, b_sage_l, W_sage_r, W_fus, b_fus):
    row = edge_index[0]
    col = edge_index[1]
    npad = E_PAD - row.shape[0]
    ar = jnp.arange(npad, dtype=jnp.int32)
    row_p = jnp.concatenate([row, (ar * 37) % N])
    col_p = jnp.concatenate([col, N + ar % (NP - N)])

    x_p = jnp.zeros((NP, D), jnp.float32).at[:N].set(x)
    att2 = jnp.stack([W_gat @ att_src, W_gat @ att_dst], axis=1)

    xw, xg, a2, xsr = _tc_pre(x_p, W_gcn, W_gat, att2, W_sage_r)

    row2 = row_p.reshape(NW * FNSTEP, FBLK)
    col2 = col_p.reshape(NW * FNSTEP, FBLK)
    ae, asum_parts, cnt_parts = _sc_edge_scalars(row_p, col_p,
                                                 a2.reshape(2 * NP))
    ae2 = ae.reshape(NW * FNSTEP, FBLK)
    s_sage = _sc_agg_sage(x_p, row2, col2)
    u = _tc_mid(cnt_parts, xw)
    s_gcn, s_gat = _sc_agg_gg(u, xg, row2, col2, ae2)

    out = _tc_post(cnt_parts, asum_parts, s_sage, s_gcn, s_gat, xsr, xw, xg,
                   a2, W_sage_l, W_fus, b_gcn, b_gat, b_sage_l, b_fus)
    return out[:N]
